# Initial kernel scaffold; baseline (speedup 1.0000x reference)
import jax
import jax.numpy as jnp
from jax.experimental import pallas as pl


def kernel(boxes, scores):
    raise NotImplementedError("write your pallas kernel here")


# speedup vs baseline: 2.6021x; 2.6021x over previous
import jax
import jax.numpy as jnp
from jax.experimental import pallas as pl
from jax.experimental.pallas import tpu as pltpu

NCLS = 80
THR = 0.05
NMS_THR = 0.5
TOPK_N = 100
M = 1000
MP = 1024
NROW = 5000
NROWP = 5120
IMG = 1024.0
OFFS = IMG + IMG


def _stage1_body(scores_ref, boxes_ref, flat_ref, bx_ref):
    s = scores_ref[...]
    m = jnp.max(s, axis=1, keepdims=True)
    e = jnp.exp(s - m)
    p = e / jnp.sum(e, axis=1, keepdims=True)
    flat_ref[...] = jnp.where(p > THR, p, -1.0)

    b = boxes_ref[...]
    cx = b[:, 0:1] * IMG
    cy = b[:, 1:2] * IMG
    w = b[:, 2:3] * (IMG * 0.25) + 4.0
    h = b[:, 3:4] * (IMG * 0.25) + 4.0
    bx_ref[:, 0:1] = jnp.clip(cx - w * 0.5, 0.0, IMG)
    bx_ref[:, 1:2] = jnp.clip(cy - h * 0.5, 0.0, IMG)
    bx_ref[:, 2:3] = jnp.clip(cx + w * 0.5, 0.0, IMG)
    bx_ref[:, 3:4] = jnp.clip(cy + h * 0.5, 0.0, IMG)


def _stage2_body(bx_ref, bxT_ref, tir_ref, tic_ref, tsc_ref,
                 kept_ref, cand_ref, clsf_ref, iou_ref):
    tir = tir_ref[...]
    tic = tic_ref[...]
    row_r = tir // NCLS
    cls_r = tir % NCLS
    row_c = tic // NCLS
    cls_c = tic % NCLS

    CH = 1024
    cand = jnp.zeros((MP, 4), jnp.float32)
    candT = jnp.zeros((4, MP), jnp.float32)
    for c in range(NROWP // CH):
        lane = jax.lax.broadcasted_iota(jnp.int32, (MP, CH), 1) + c * CH
        oh = (lane == row_r).astype(jnp.float32)
        cand = cand + jnp.dot(oh, bx_ref[c * CH:(c + 1) * CH, :],
                              preferred_element_type=jnp.float32)
        sub = jax.lax.broadcasted_iota(jnp.int32, (CH, MP), 0) + c * CH
        ohT = (sub == row_c).astype(jnp.float32)
        candT = candT + jnp.dot(bxT_ref[:, c * CH:(c + 1) * CH], ohT,
                                preferred_element_type=jnp.float32)
    cand_ref[...] = cand
    clsf_ref[...] = cls_r.astype(jnp.float32)

    off_r = cls_r.astype(jnp.float32) * OFFS
    off_c = cls_c.astype(jnp.float32) * OFFS
    x1c = candT[0:1, :] + off_c
    y1c = candT[1:2, :] + off_c
    x2c = candT[2:3, :] + off_c
    y2c = candT[3:4, :] + off_c
    area_c = (x2c - x1c) * (y2c - y1c)

    RB = 128
    for rb in range(MP // RB):
        sl = slice(rb * RB, (rb + 1) * RB)
        x1r = cand[sl, 0:1] + off_r[sl]
        y1r = cand[sl, 1:2] + off_r[sl]
        x2r = cand[sl, 2:3] + off_r[sl]
        y2r = cand[sl, 3:4] + off_r[sl]
        area_r = (x2r - x1r) * (y2r - y1r)
        ww = jnp.maximum(jnp.minimum(x2r, x2c) - jnp.maximum(x1r, x1c), 0.0)
        hh = jnp.maximum(jnp.minimum(y2r, y2c) - jnp.maximum(y1r, y1c), 0.0)
        inter = ww * hh
        union = area_r + area_c - inter
        iou_ref[sl, :] = inter / jnp.maximum(union, 1e-9)

    ts = tsc_ref[...]
    lane_i = jax.lax.broadcasted_iota(jnp.int32, (1, MP), 1)
    keep0 = jnp.where(ts > THR, 1.0, 0.0)

    def body(i, keep):
        iou_i = iou_ref[pl.ds(i, 1), :]
        keep_i = jnp.sum(jnp.where(lane_i == i, keep, 0.0))
        supf = jnp.where((iou_i > NMS_THR) & (lane_i > i), keep_i, 0.0)
        return keep * (1.0 - supf)

    keep = jax.lax.fori_loop(0, M, body, keep0)
    kept_ref[...] = jnp.where(keep > 0.5, ts, -1.0)


def _stage3_body(cand_ref, clsf_ref, fi_ref, fs_ref, ob_ref, os_ref, oc_ref):
    fi = fi_ref[...]
    fs = fs_ref[...]
    lane = jax.lax.broadcasted_iota(jnp.int32, (128, MP), 1)
    oh = (lane == fi).astype(jnp.float32)
    selb = jnp.dot(oh, cand_ref[...], preferred_element_type=jnp.float32)
    selc = jnp.dot(oh, clsf_ref[...], preferred_element_type=jnp.float32)
    valid = fs > THR
    vf = valid.astype(jnp.float32)
    ob_ref[...] = selb * vf
    os_ref[...] = fs * vf
    oc_ref[...] = jnp.where(valid, selc, -1.0)


@jax.jit
def kernel(boxes, scores):
    scores_p = jnp.pad(scores, ((0, NROWP - NROW), (0, 128 - 81)),
                       constant_values=-1e30)
    boxes_p = jnp.pad(boxes, ((0, NROWP - NROW), (0, 0)))
    flat_p, bx = pl.pallas_call(
        _stage1_body,
        grid=(10,),
        in_specs=[pl.BlockSpec((512, 128), lambda i: (i, 0)),
                  pl.BlockSpec((512, 4), lambda i: (i, 0))],
        out_specs=[pl.BlockSpec((512, 128), lambda i: (i, 0)),
                   pl.BlockSpec((512, 4), lambda i: (i, 0))],
        out_shape=[jax.ShapeDtypeStruct((NROWP, 128), jnp.float32),
                   jax.ShapeDtypeStruct((NROWP, 4), jnp.float32)],
    )(scores_p, boxes_p)

    flat = flat_p[:NROW, :NCLS].reshape(-1)
    top_scores, top_idx = jax.lax.top_k(flat, M)
    ti = jnp.pad(top_idx, (0, MP - M))
    ts = jnp.pad(top_scores, (0, MP - M), constant_values=-1.0)

    kept, cand, clsf = pl.pallas_call(
        _stage2_body,
        out_shape=[jax.ShapeDtypeStruct((1, MP), jnp.float32),
                   jax.ShapeDtypeStruct((MP, 4), jnp.float32),
                   jax.ShapeDtypeStruct((MP, 1), jnp.float32)],
        scratch_shapes=[pltpu.VMEM((MP, MP), jnp.float32)],
    )(bx, bx.T, ti.reshape(MP, 1), ti.reshape(1, MP), ts.reshape(1, MP))

    kept_scores = kept.reshape(MP)[:M]
    fs, fi = jax.lax.top_k(kept_scores, TOPK_N)
    fi_p = jnp.pad(fi, (0, 128 - TOPK_N)).reshape(128, 1)
    fs_p = jnp.pad(fs, (0, 128 - TOPK_N), constant_values=-1.0).reshape(128, 1)

    ob, osc, ocl = pl.pallas_call(
        _stage3_body,
        out_shape=[jax.ShapeDtypeStruct((128, 4), jnp.float32),
                   jax.ShapeDtypeStruct((128, 1), jnp.float32),
                   jax.ShapeDtypeStruct((128, 1), jnp.float32)],
    )(cand, clsf, fi_p, fs_p)
    return ob[:TOPK_N], osc[:TOPK_N, 0], ocl[:TOPK_N, 0].astype(jnp.int32)


# speedup vs baseline: 3.0843x; 1.1853x over previous
import jax
import jax.numpy as jnp
from jax.experimental import pallas as pl
from jax.experimental.pallas import tpu as pltpu

NCLS = 80
THR = 0.05
NMS_THR = 0.5
TOPK_N = 100
M = 1000
MP = 1024
NROW = 5000
NROWP = 5120
IMG = 1024.0
OFFS = IMG + IMG


def _stage1_body(scores_ref, boxes_ref, flat_ref, bx_ref):
    s = scores_ref[...]
    m = jnp.max(s, axis=1, keepdims=True)
    e = jnp.exp(s - m)
    p = e / jnp.sum(e, axis=1, keepdims=True)
    flat_ref[...] = jnp.where(p > THR, p, -1.0)

    b = boxes_ref[...]
    cx = b[:, 0:1] * IMG
    cy = b[:, 1:2] * IMG
    w = b[:, 2:3] * (IMG * 0.25) + 4.0
    h = b[:, 3:4] * (IMG * 0.25) + 4.0
    bx_ref[:, 0:1] = jnp.clip(cx - w * 0.5, 0.0, IMG)
    bx_ref[:, 1:2] = jnp.clip(cy - h * 0.5, 0.0, IMG)
    bx_ref[:, 2:3] = jnp.clip(cx + w * 0.5, 0.0, IMG)
    bx_ref[:, 3:4] = jnp.clip(cy + h * 0.5, 0.0, IMG)


def _stage2_body(bx_ref, bxT_ref, tir_ref, tic_ref, tsc_ref,
                 kept_ref, cand_ref, clsf_ref, iou_ref):
    tir = tir_ref[...]
    tic = tic_ref[...]
    row_r = tir // NCLS
    cls_r = tir % NCLS
    row_c = tic // NCLS
    cls_c = tic % NCLS

    CH = 1024
    cand = jnp.zeros((MP, 4), jnp.float32)
    candT = jnp.zeros((4, MP), jnp.float32)
    for c in range(NROWP // CH):
        lane = jax.lax.broadcasted_iota(jnp.int32, (MP, CH), 1) + c * CH
        oh = (lane == row_r).astype(jnp.float32)
        cand = cand + jnp.dot(oh, bx_ref[c * CH:(c + 1) * CH, :],
                              preferred_element_type=jnp.float32)
        sub = jax.lax.broadcasted_iota(jnp.int32, (CH, MP), 0) + c * CH
        ohT = (sub == row_c).astype(jnp.float32)
        candT = candT + jnp.dot(bxT_ref[:, c * CH:(c + 1) * CH], ohT,
                                preferred_element_type=jnp.float32)
    cand_ref[...] = cand
    clsf_ref[...] = cls_r.astype(jnp.float32)

    off_r = cls_r.astype(jnp.float32) * OFFS
    off_c = cls_c.astype(jnp.float32) * OFFS
    x1c = candT[0:1, :] + off_c
    y1c = candT[1:2, :] + off_c
    x2c = candT[2:3, :] + off_c
    y2c = candT[3:4, :] + off_c
    area_c = (x2c - x1c) * (y2c - y1c)

    RB = 128
    for rb in range(MP // RB):
        sl = slice(rb * RB, (rb + 1) * RB)
        x1r = cand[sl, 0:1] + off_r[sl]
        y1r = cand[sl, 1:2] + off_r[sl]
        x2r = cand[sl, 2:3] + off_r[sl]
        y2r = cand[sl, 3:4] + off_r[sl]
        area_r = (x2r - x1r) * (y2r - y1r)
        ww = jnp.maximum(jnp.minimum(x2r, x2c) - jnp.maximum(x1r, x1c), 0.0)
        hh = jnp.maximum(jnp.minimum(y2r, y2c) - jnp.maximum(y1r, y1c), 0.0)
        inter = ww * hh
        union = area_r + area_c - inter
        iou = inter / jnp.maximum(union, 1e-9)
        rows = jax.lax.broadcasted_iota(jnp.int32, (RB, MP), 0) + rb * RB
        cols = jax.lax.broadcasted_iota(jnp.int32, (RB, MP), 1)
        iou_ref[sl, :] = jnp.where((iou > NMS_THR) & (rows < cols), 1.0, 0.0)

    ts = tsc_ref[...]
    keep0 = jnp.where(ts > THR, 1.0, 0.0)

    def cond(carry):
        return carry[1] > 0.5

    def body(carry):
        keep, _ = carry
        supc = jnp.dot(keep, iou_ref[...], preferred_element_type=jnp.float32)
        keep_new = jnp.where(supc > 0.5, 0.0, keep0)
        delta = jnp.sum(jnp.abs(keep_new - keep))
        return (keep_new, delta)

    keep, _ = jax.lax.while_loop(cond, body, (keep0, jnp.float32(1.0)))
    kept_ref[...] = jnp.where(keep > 0.5, ts, -1.0)


def _stage3_body(cand_ref, clsf_ref, fi_ref, fs_ref, ob_ref, os_ref, oc_ref):
    fi = fi_ref[...]
    fs = fs_ref[...]
    lane = jax.lax.broadcasted_iota(jnp.int32, (128, MP), 1)
    oh = (lane == fi).astype(jnp.float32)
    selb = jnp.dot(oh, cand_ref[...], preferred_element_type=jnp.float32)
    selc = jnp.dot(oh, clsf_ref[...], preferred_element_type=jnp.float32)
    valid = fs > THR
    vf = valid.astype(jnp.float32)
    ob_ref[...] = selb * vf
    os_ref[...] = fs * vf
    oc_ref[...] = jnp.where(valid, selc, -1.0)


@jax.jit
def kernel(boxes, scores):
    scores_p = jnp.pad(scores, ((0, NROWP - NROW), (0, 128 - 81)),
                       constant_values=-1e30)
    boxes_p = jnp.pad(boxes, ((0, NROWP - NROW), (0, 0)))
    flat_p, bx = pl.pallas_call(
        _stage1_body,
        grid=(10,),
        in_specs=[pl.BlockSpec((512, 128), lambda i: (i, 0)),
                  pl.BlockSpec((512, 4), lambda i: (i, 0))],
        out_specs=[pl.BlockSpec((512, 128), lambda i: (i, 0)),
                   pl.BlockSpec((512, 4), lambda i: (i, 0))],
        out_shape=[jax.ShapeDtypeStruct((NROWP, 128), jnp.float32),
                   jax.ShapeDtypeStruct((NROWP, 4), jnp.float32)],
    )(scores_p, boxes_p)

    flat = flat_p[:NROW, :NCLS].reshape(-1)
    top_scores, top_idx = jax.lax.top_k(flat, M)
    ti = jnp.pad(top_idx, (0, MP - M))
    ts = jnp.pad(top_scores, (0, MP - M), constant_values=-1.0)

    kept, cand, clsf = pl.pallas_call(
        _stage2_body,
        out_shape=[jax.ShapeDtypeStruct((1, MP), jnp.float32),
                   jax.ShapeDtypeStruct((MP, 4), jnp.float32),
                   jax.ShapeDtypeStruct((MP, 1), jnp.float32)],
        scratch_shapes=[pltpu.VMEM((MP, MP), jnp.float32)],
    )(bx, bx.T, ti.reshape(MP, 1), ti.reshape(1, MP), ts.reshape(1, MP))

    kept_scores = kept.reshape(MP)[:M]
    fs, fi = jax.lax.top_k(kept_scores, TOPK_N)
    fi_p = jnp.pad(fi, (0, 128 - TOPK_N)).reshape(128, 1)
    fs_p = jnp.pad(fs, (0, 128 - TOPK_N), constant_values=-1.0).reshape(128, 1)

    ob, osc, ocl = pl.pallas_call(
        _stage3_body,
        out_shape=[jax.ShapeDtypeStruct((128, 4), jnp.float32),
                   jax.ShapeDtypeStruct((128, 1), jnp.float32),
                   jax.ShapeDtypeStruct((128, 1), jnp.float32)],
    )(cand, clsf, fi_p, fs_p)
    return ob[:TOPK_N], osc[:TOPK_N, 0], ocl[:TOPK_N, 0].astype(jnp.int32)
